# baseline (speedup 1.0000x reference)
import functools

import jax
import jax.numpy as jnp
from jax import lax
from jax.experimental import pallas as pl
from jax.experimental.pallas import tpu as pltpu
from jax.experimental.pallas import tpu_sc as plsc

B = 1024
N = 50000
E = 800000
NCLS_A = 16
NCLS_C = 6
NCLS_B = 5

NC = 2
NS = 16
NW = NC * NS

EPW = E // NW
CH = 1000
NCHUNK = EPW // CH
SCAT1 = 8
ATAB_ROWS = 50048
A_JUNK = N
ROWS1_PT = ATAB_ROWS // NS

NPW = 1568
SCAT2 = 13
GTAB_ROWS = 1152
G_JUNK = B
ROWS2_PT = GTAB_ROWS // NS

LN2 = 0.6931471805599453


def _ln(s):
    bits = plsc.bitcast(s, jnp.int32)
    e = (bits >> 23) - 127
    m = plsc.bitcast((bits & 0x007FFFFF) | 0x3F800000, jnp.float32)
    z = (m - 1.0) / (m + 1.0)
    z2 = z * z
    p = 1.0 / 7.0 + z2 * (1.0 / 9.0)
    p = 1.0 / 5.0 + z2 * p
    p = 1.0 / 3.0 + z2 * p
    atanh = z * (1.0 + z2 * p)
    return e.astype(jnp.float32) * LN2 + 2.0 * atanh


def _lane():
    return lax.iota(jnp.int32, 16)


def _ce_flat(buf, i_loc, t, ncls, stride=None):
    base = i_loc * (stride or ncls)
    xs = [plsc.load_gather(buf, [base + c]) for c in range(ncls)]
    m = xs[0]
    for x in xs[1:]:
        m = jnp.maximum(m, x)
    s = jnp.exp(xs[0] - m)
    for x in xs[1:]:
        s = s + jnp.exp(x - m)
    xt = plsc.load_gather(buf, [base + t])
    return _ln(s) + m - xt


def _fill(buf, n, value):
    def body(k, _):
        buf[pl.ds(k * 16, 16)] = jnp.full((16,), value, jnp.float32)
        return 0
    lax.fori_loop(0, n // 16, body, 0)


def _k1_body(pb_hbm, tb_hbm, ai_hbm, out_hbm,
             lg_v, tb_v, ai_v, val_v, idx_v, zb_v,
             atab_s, atab_c, sem):
    cid = lax.axis_index("c")
    sid = lax.axis_index("s")
    wid = sid * NC + cid
    ebase = wid * EPW

    _fill(zb_v, ROWS1_PT + 8, 0.0)
    pltpu.sync_copy(zb_v.at[pl.ds(0, ROWS1_PT)],
                    atab_s.at[pl.ds(sid * ROWS1_PT, ROWS1_PT)])
    pltpu.sync_copy(zb_v.at[pl.ds(0, ROWS1_PT)],
                    atab_c.at[pl.ds(sid * ROWS1_PT, ROWS1_PT)])
    _fill(zb_v, SCAT1 * 128, 1.0)
    jnk = jnp.full((16,), A_JUNK, dtype=jnp.int32)
    for col in range((CH % 128 + 15) // 16 * 16, 128, 16):
        idx_v[CH // 128, pl.ds(col, 16)] = jnk
    plsc.subcore_barrier()

    def chunk_body(j, _):
        off = ebase + j * CH
        pltpu.sync_copy(pb_hbm.at[pl.ds(off * 8, CH * 8)], lg_v)
        pltpu.sync_copy(tb_hbm.at[pl.ds(off, CH)], tb_v.at[pl.ds(0, CH)])
        pltpu.sync_copy(ai_hbm.at[pl.ds(off, CH)], ai_v.at[pl.ds(0, CH)])

        for q in range(SCAT1):
            nvr = 8 if (q + 1) * 128 <= CH else (CH - q * 128 + 15) // 16

            def vreg_body(kk, _, q=q):
                col = kk * 16
                f = q * 128 + col
                i_loc = f + _lane()
                valid = i_loc < CH
                i_safe = jnp.minimum(i_loc, CH - 1)
                t = jnp.where(valid, tb_v[pl.ds(f, 16)], 0)
                ce = _ce_flat(lg_v, i_safe, t, NCLS_B, 8)
                g = ai_v[pl.ds(f, 16)]
                dest = jnp.where(valid, g, A_JUNK)
                idx_v[q, pl.ds(col, 16)] = dest
                val_v[pl.ds(f, 16)] = ce
                return 0

            lax.fori_loop(0, nvr, vreg_body, 0)

        copies = []
        for q in range(SCAT1):
            copies.append(pltpu.async_copy(
                val_v.at[pl.ds(q * 128, 128)],
                atab_s.at[idx_v.at[q]], sem, add=True))
            copies.append(pltpu.async_copy(
                zb_v.at[pl.ds(q * 128, 128)],
                atab_c.at[idx_v.at[q]], sem, add=True))
        for c in copies:
            c.wait()
        return 0

    lax.fori_loop(0, NCHUNK, chunk_body, 0)
    plsc.subcore_barrier()
    pltpu.sync_copy(atab_s.at[pl.ds(sid * ROWS1_PT, ROWS1_PT)],
                    zb_v.at[pl.ds(0, ROWS1_PT)])
    pltpu.sync_copy(
        zb_v.at[pl.ds(0, ROWS1_PT)],
        out_hbm.at[pl.ds(cid * 2 * ATAB_ROWS + sid * ROWS1_PT, ROWS1_PT)])
    pltpu.sync_copy(atab_c.at[pl.ds(sid * ROWS1_PT, ROWS1_PT)],
                    zb_v.at[pl.ds(0, ROWS1_PT)])
    pltpu.sync_copy(
        zb_v.at[pl.ds(0, ROWS1_PT)],
        out_hbm.at[pl.ds((cid * 2 + 1) * ATAB_ROWS + sid * ROWS1_PT,
                         ROWS1_PT)])


def _k2_body(pc_hbm, tc_hbm, pa_hbm, ta_hbm, pch_hbm, tch_hbm, bt_hbm,
             bp_hbm, out_hbm,
             pc_v, tc_v, pa_v, ta_v, pch_v, tch_v, bt_v,
             bp0s_v, bp0c_v, bp1s_v, bp1c_v,
             r_v, a_v, c_v, b_v, ones_v, idx_v, zb_v, gtabs, sem):
    cid = lax.axis_index("c")
    sid = lax.axis_index("s")
    wid = sid * NC + cid
    vstart = wid * NPW
    base = jnp.minimum(vstart, N - NPW)

    _fill(zb_v, 80, 0.0)
    for tbl in gtabs:
        pltpu.sync_copy(zb_v.at[pl.ds(0, ROWS2_PT)],
                        tbl.at[pl.ds(sid * ROWS2_PT, ROWS2_PT)])
    _fill(ones_v, SCAT2 * 128, 1.0)
    jnk = jnp.full((16,), G_JUNK, dtype=jnp.int32)
    for col in range(NPW % 128, 128, 16):
        idx_v[NPW // 128, pl.ds(col, 16)] = jnk
    plsc.subcore_barrier()

    copies = [
        pltpu.async_copy(pc_hbm.at[pl.ds(base * 3, NPW * 3)], pc_v, sem),
        pltpu.async_copy(tc_hbm.at[pl.ds(base * 3, NPW * 3)], tc_v, sem),
        pltpu.async_copy(pa_hbm.at[pl.ds(base * NCLS_A, NPW * NCLS_A)],
                         pa_v, sem),
        pltpu.async_copy(ta_hbm.at[pl.ds(base, NPW)], ta_v, sem),
        pltpu.async_copy(pch_hbm.at[pl.ds(base * NCLS_C, NPW * NCLS_C)],
                         pch_v, sem),
        pltpu.async_copy(tch_hbm.at[pl.ds(base, NPW)], tch_v, sem),
        pltpu.async_copy(bt_hbm.at[pl.ds(base, NPW)], bt_v, sem),
        pltpu.async_copy(bp_hbm.at[pl.ds(base, NPW)], bp0s_v, sem),
        pltpu.async_copy(bp_hbm.at[pl.ds(ATAB_ROWS + base, NPW)],
                         bp0c_v, sem),
        pltpu.async_copy(bp_hbm.at[pl.ds(2 * ATAB_ROWS + base, NPW)],
                         bp1s_v, sem),
        pltpu.async_copy(bp_hbm.at[pl.ds(3 * ATAB_ROWS + base, NPW)],
                         bp1c_v, sem),
    ]
    for c in copies:
        c.wait()

    for q in range(SCAT2):
        nvr = 8 if (q + 1) * 128 <= NPW else (NPW - q * 128 + 15) // 16

        def vreg_body(kk, _, q=q):
            col = kk * 16
            f = q * 128 + col
            i_loc = f + _lane()
            acc = jnp.zeros((16,), jnp.float32)
            for d in range(3):
                pd = plsc.load_gather(pc_v, [i_loc * 3 + d])
                td = plsc.load_gather(tc_v, [i_loc * 3 + d])
                acc = acc + (pd - td) * (pd - td)
            ta = ta_v[pl.ds(f, 16)]
            ca = _ce_flat(pa_v, i_loc, ta, NCLS_A)
            tch = tch_v[pl.ds(f, 16)]
            cc = _ce_flat(pch_v, i_loc, tch, NCLS_C)
            bsum = bp0s_v[pl.ds(f, 16)] + bp1s_v[pl.ds(f, 16)]
            bcnt = bp0c_v[pl.ds(f, 16)] + bp1c_v[pl.ds(f, 16)]
            bv = 0.5 * bsum / jnp.maximum(bcnt, 1.0)
            g = bt_v[pl.ds(f, 16)]
            dest = jnp.where(base + i_loc >= vstart, g, G_JUNK)
            idx_v[q, pl.ds(col, 16)] = dest
            r_v[pl.ds(f, 16)] = acc * (1.0 / 3.0)
            a_v[pl.ds(f, 16)] = ca
            c_v[pl.ds(f, 16)] = cc
            b_v[pl.ds(f, 16)] = bv
            return 0

        lax.fori_loop(0, nvr, vreg_body, 0)

    srcs = [r_v, a_v, c_v, b_v, ones_v]
    copies = []
    for q in range(SCAT2):
        for src, tbl in zip(srcs, gtabs):
            copies.append(pltpu.async_copy(
                src.at[pl.ds(q * 128, 128)],
                tbl.at[idx_v.at[q]], sem, add=True))
    for c in copies:
        c.wait()
    plsc.subcore_barrier()
    for tn, tbl in enumerate(gtabs):
        pltpu.sync_copy(tbl.at[pl.ds(sid * ROWS2_PT, ROWS2_PT)],
                        zb_v.at[pl.ds(0, ROWS2_PT)])
        pltpu.sync_copy(
            zb_v.at[pl.ds(0, ROWS2_PT)],
            out_hbm.at[pl.ds((cid * 5 + tn) * GTAB_ROWS + sid * ROWS2_PT,
                             ROWS2_PT)])


def _k3_body(gp_hbm, w_hbm, out_hbm, t0_v, t1_v, cnt_v, w_v, ob_v, sem):
    cid = lax.axis_index("c")
    sid = lax.axis_index("s")

    @pl.when(jnp.logical_and(cid == 0, sid == 0))
    def _():
        o1 = 5 * GTAB_ROWS
        pltpu.async_copy(gp_hbm.at[pl.ds(4 * GTAB_ROWS, B)], t0_v, sem).wait()
        pltpu.async_copy(gp_hbm.at[pl.ds(o1 + 4 * GTAB_ROWS, B)],
                         t1_v, sem).wait()
        pltpu.async_copy(w_hbm, w_v, sem).wait()

        def cnt_body(k, _):
            f = k * 16
            cnt_v[pl.ds(f, 16)] = jnp.maximum(
                t0_v[pl.ds(f, 16)] + t1_v[pl.ds(f, 16)], 1.0)
            return 0

        lax.fori_loop(0, B // 16, cnt_body, 0)

        lane = _lane()
        out = jnp.zeros((16,), jnp.float32)
        for tbl in range(4):
            pltpu.async_copy(gp_hbm.at[pl.ds(tbl * GTAB_ROWS, B)],
                             t0_v, sem).wait()
            pltpu.async_copy(gp_hbm.at[pl.ds(o1 + tbl * GTAB_ROWS, B)],
                             t1_v, sem).wait()

            def body(k, acc):
                f = k * 16
                tot = t0_v[pl.ds(f, 16)] + t1_v[pl.ds(f, 16)]
                return acc + w_v[pl.ds(f, 16)] * tot / cnt_v[pl.ds(f, 16)]

            acc = lax.fori_loop(0, B // 16, body, jnp.zeros((16,),
                                                            jnp.float32))
            li = jnp.sum(acc, axis=0)
            out = out + jnp.where(lane == tbl, li, 0.0)
        ob_v[...] = out
        pltpu.sync_copy(ob_v, out_hbm)


_mesh = plsc.VectorSubcoreMesh(core_axis_name="c", subcore_axis_name="s")
_cparams = pltpu.CompilerParams(needs_layout_passes=False)

_k1 = functools.partial(
    pl.kernel,
    out_type=jax.ShapeDtypeStruct((NC * 2 * ATAB_ROWS,), jnp.float32),
    mesh=_mesh,
    compiler_params=_cparams,
    scratch_types=[
        pltpu.VMEM((CH * 8,), jnp.float32),
        pltpu.VMEM((SCAT1 * 128,), jnp.int32),
        pltpu.VMEM((SCAT1 * 128,), jnp.int32),
        pltpu.VMEM((SCAT1 * 128,), jnp.float32),
        pltpu.VMEM((SCAT1, 128), jnp.int32),
        pltpu.VMEM((ROWS1_PT + 8,), jnp.float32),
        pltpu.VMEM_SHARED((ATAB_ROWS,), jnp.float32),
        pltpu.VMEM_SHARED((ATAB_ROWS,), jnp.float32),
        pltpu.SemaphoreType.DMA,
    ],
)(_k1_body)

_k2 = functools.partial(
    pl.kernel,
    out_type=jax.ShapeDtypeStruct((NC * 5 * GTAB_ROWS,), jnp.float32),
    mesh=_mesh,
    compiler_params=_cparams,
    scratch_types=[
        pltpu.VMEM((NPW * 3,), jnp.float32),
        pltpu.VMEM((NPW * 3,), jnp.float32),
        pltpu.VMEM((NPW * NCLS_A,), jnp.float32),
        pltpu.VMEM((NPW,), jnp.int32),
        pltpu.VMEM((NPW * NCLS_C,), jnp.float32),
        pltpu.VMEM((NPW,), jnp.int32),
        pltpu.VMEM((NPW,), jnp.int32),
        pltpu.VMEM((NPW,), jnp.float32),
        pltpu.VMEM((NPW,), jnp.float32),
        pltpu.VMEM((NPW,), jnp.float32),
        pltpu.VMEM((NPW,), jnp.float32),
        pltpu.VMEM((SCAT2 * 128,), jnp.float32),
        pltpu.VMEM((SCAT2 * 128,), jnp.float32),
        pltpu.VMEM((SCAT2 * 128,), jnp.float32),
        pltpu.VMEM((SCAT2 * 128,), jnp.float32),
        pltpu.VMEM((SCAT2 * 128,), jnp.float32),
        pltpu.VMEM((SCAT2, 128), jnp.int32),
        pltpu.VMEM((80,), jnp.float32),
        [pltpu.VMEM_SHARED((GTAB_ROWS,), jnp.float32) for _ in range(5)],
        pltpu.SemaphoreType.DMA,
    ],
)(_k2_body)

_k3 = functools.partial(
    pl.kernel,
    out_type=jax.ShapeDtypeStruct((16,), jnp.float32),
    mesh=_mesh,
    compiler_params=_cparams,
    scratch_types=[
        pltpu.VMEM((B,), jnp.float32),
        pltpu.VMEM((B,), jnp.float32),
        pltpu.VMEM((B,), jnp.float32),
        pltpu.VMEM((B,), jnp.float32),
        pltpu.VMEM((16,), jnp.float32),
        pltpu.SemaphoreType.DMA,
    ],
)(_k3_body)


def kernel(pred_coords, true_coords, pred_atoms, true_atoms, pred_charges,
           true_charges, pred_bonds, true_bonds, batch,
           bond_aggregation_index, weights):
    pb8 = jnp.pad(pred_bonds, ((0, 0), (0, 3))).reshape(-1)
    atab = _k1(pb8, true_bonds, bond_aggregation_index)
    gtab = _k2(pred_coords.reshape(-1), true_coords.reshape(-1),
               pred_atoms.reshape(-1), true_atoms,
               pred_charges.reshape(-1), true_charges, batch, atab)
    out = _k3(gtab, weights)
    return out[:4]


# speedup vs baseline: 3.2227x; 3.2227x over previous
import functools

import jax
import jax.numpy as jnp
from jax import lax
from jax.experimental import pallas as pl
from jax.experimental.pallas import tpu as pltpu
from jax.experimental.pallas import tpu_sc as plsc

B = 1024
N = 50000
E = 800000
NCLS_A = 16
NCLS_C = 6
NCLS_B = 5

NC = 2
NS = 16
NW = NC * NS

EPW = E // NW
CH = 1000
NCHUNK = EPW // CH
CHP = CH + 16
SCAT1 = 8
ATAB_ROWS = 50048
A_JUNK = N
ROWS1_PT = ATAB_ROWS // NS

NPW = 1568
SCAT2 = 13
GTAB_ROWS = 1152
G_JUNK = B
ROWS2_PT = GTAB_ROWS // NS

LN2 = 0.6931471805599453


def _ln(s):
    bits = plsc.bitcast(s, jnp.int32)
    e = (bits >> 23) - 127
    m = plsc.bitcast((bits & 0x007FFFFF) | 0x3F800000, jnp.float32)
    z = (m - 1.0) / (m + 1.0)
    z2 = z * z
    p = 1.0 / 7.0 + z2 * (1.0 / 9.0)
    p = 1.0 / 5.0 + z2 * p
    p = 1.0 / 3.0 + z2 * p
    atanh = z * (1.0 + z2 * p)
    return e.astype(jnp.float32) * LN2 + 2.0 * atanh


def _lane():
    return lax.iota(jnp.int32, 16)


def _ce_flat(buf, i_loc, t, ncls, stride=None):
    base = i_loc * (stride or ncls)
    xs = [plsc.load_gather(buf, [base + c]) for c in range(ncls)]
    m = xs[0]
    for x in xs[1:]:
        m = jnp.maximum(m, x)
    s = jnp.exp(xs[0] - m)
    for x in xs[1:]:
        s = s + jnp.exp(x - m)
    xt = plsc.load_gather(buf, [base + t])
    return _ln(s) + m - xt


def _fill(buf, n, value):
    def body(k, _):
        buf[pl.ds(k * 16, 16)] = jnp.full((16,), value, jnp.float32)
        return 0
    lax.fori_loop(0, n // 16, body, 0)


def _k1_body(pb_hbm, tb_hbm, ai_hbm, out_hbm,
             lg_v, tb_v, ai_v, val_v, idx_v, zb_v,
             atab_s, atab_c, sem):
    cid = lax.axis_index("c")
    sid = lax.axis_index("s")
    wid = sid * NC + cid
    ebase = wid * EPW

    _fill(zb_v, ROWS1_PT + 8, 0.0)
    pltpu.sync_copy(zb_v.at[pl.ds(0, ROWS1_PT)],
                    atab_s.at[pl.ds(sid * ROWS1_PT, ROWS1_PT)])
    pltpu.sync_copy(zb_v.at[pl.ds(0, ROWS1_PT)],
                    atab_c.at[pl.ds(sid * ROWS1_PT, ROWS1_PT)])
    _fill(zb_v, SCAT1 * 128, 1.0)
    jnk = jnp.full((16,), A_JUNK, dtype=jnp.int32)
    for col in range((CH % 128 + 15) // 16 * 16, 128, 16):
        idx_v[CH // 128, pl.ds(col, 16)] = jnk
    plsc.subcore_barrier()

    def chunk_body(j, _):
        off = ebase + j * CH
        for c in range(NCLS_B):
            pltpu.sync_copy(pb_hbm.at[pl.ds(c * E + off, CH)],
                            lg_v.at[pl.ds(c * CHP, CH)])
        pltpu.sync_copy(tb_hbm.at[pl.ds(off, CH)], tb_v.at[pl.ds(0, CH)])
        pltpu.sync_copy(ai_hbm.at[pl.ds(off, CH)], ai_v.at[pl.ds(0, CH)])

        for q in range(SCAT1):
            nvr = 8 if (q + 1) * 128 <= CH else (CH - q * 128 + 15) // 16

            def vreg_body(kk, _, q=q):
                col = kk * 16
                f = q * 128 + col
                i_loc = f + _lane()
                valid = i_loc < CH
                t = tb_v[pl.ds(f, 16)]
                xs = [lg_v[pl.ds(c * CHP + f, 16)] for c in range(NCLS_B)]
                m = xs[0]
                for x in xs[1:]:
                    m = jnp.maximum(m, x)
                s = jnp.exp(xs[0] - m)
                for x in xs[1:]:
                    s = s + jnp.exp(x - m)
                xt = xs[0]
                for c in range(1, NCLS_B):
                    xt = jnp.where(t == c, xs[c], xt)
                ce = _ln(s) + m - xt
                g = ai_v[pl.ds(f, 16)]
                dest = jnp.where(valid, g, A_JUNK)
                idx_v[q, pl.ds(col, 16)] = dest
                val_v[pl.ds(f, 16)] = ce
                return 0

            lax.fori_loop(0, nvr, vreg_body, 0)

        copies = []
        for q in range(SCAT1):
            copies.append(pltpu.async_copy(
                val_v.at[pl.ds(q * 128, 128)],
                atab_s.at[idx_v.at[q]], sem, add=True))
            copies.append(pltpu.async_copy(
                zb_v.at[pl.ds(q * 128, 128)],
                atab_c.at[idx_v.at[q]], sem, add=True))
        for c in copies:
            c.wait()
        return 0

    lax.fori_loop(0, NCHUNK, chunk_body, 0)
    plsc.subcore_barrier()
    pltpu.sync_copy(atab_s.at[pl.ds(sid * ROWS1_PT, ROWS1_PT)],
                    zb_v.at[pl.ds(0, ROWS1_PT)])
    pltpu.sync_copy(
        zb_v.at[pl.ds(0, ROWS1_PT)],
        out_hbm.at[pl.ds(cid * 2 * ATAB_ROWS + sid * ROWS1_PT, ROWS1_PT)])
    pltpu.sync_copy(atab_c.at[pl.ds(sid * ROWS1_PT, ROWS1_PT)],
                    zb_v.at[pl.ds(0, ROWS1_PT)])
    pltpu.sync_copy(
        zb_v.at[pl.ds(0, ROWS1_PT)],
        out_hbm.at[pl.ds((cid * 2 + 1) * ATAB_ROWS + sid * ROWS1_PT,
                         ROWS1_PT)])


def _k2_body(pc_hbm, tc_hbm, pa_hbm, ta_hbm, pch_hbm, tch_hbm, bt_hbm,
             bp_hbm, out_hbm,
             pc_v, tc_v, pa_v, ta_v, pch_v, tch_v, bt_v,
             bp0s_v, bp0c_v, bp1s_v, bp1c_v,
             r_v, a_v, c_v, b_v, ones_v, idx_v, zb_v, gtabs, sem):
    cid = lax.axis_index("c")
    sid = lax.axis_index("s")
    wid = sid * NC + cid
    vstart = wid * NPW
    base = jnp.minimum(vstart, N - NPW)

    _fill(zb_v, 80, 0.0)
    for tbl in gtabs:
        pltpu.sync_copy(zb_v.at[pl.ds(0, ROWS2_PT)],
                        tbl.at[pl.ds(sid * ROWS2_PT, ROWS2_PT)])
    _fill(ones_v, SCAT2 * 128, 1.0)
    jnk = jnp.full((16,), G_JUNK, dtype=jnp.int32)
    for col in range(NPW % 128, 128, 16):
        idx_v[NPW // 128, pl.ds(col, 16)] = jnk
    plsc.subcore_barrier()

    copies = [
        pltpu.async_copy(pc_hbm.at[pl.ds(base * 3, NPW * 3)], pc_v, sem),
        pltpu.async_copy(tc_hbm.at[pl.ds(base * 3, NPW * 3)], tc_v, sem),
        pltpu.async_copy(pa_hbm.at[pl.ds(base * NCLS_A, NPW * NCLS_A)],
                         pa_v, sem),
        pltpu.async_copy(ta_hbm.at[pl.ds(base, NPW)], ta_v, sem),
        pltpu.async_copy(pch_hbm.at[pl.ds(base * NCLS_C, NPW * NCLS_C)],
                         pch_v, sem),
        pltpu.async_copy(tch_hbm.at[pl.ds(base, NPW)], tch_v, sem),
        pltpu.async_copy(bt_hbm.at[pl.ds(base, NPW)], bt_v, sem),
        pltpu.async_copy(bp_hbm.at[pl.ds(base, NPW)], bp0s_v, sem),
        pltpu.async_copy(bp_hbm.at[pl.ds(ATAB_ROWS + base, NPW)],
                         bp0c_v, sem),
        pltpu.async_copy(bp_hbm.at[pl.ds(2 * ATAB_ROWS + base, NPW)],
                         bp1s_v, sem),
        pltpu.async_copy(bp_hbm.at[pl.ds(3 * ATAB_ROWS + base, NPW)],
                         bp1c_v, sem),
    ]
    for c in copies:
        c.wait()

    for q in range(SCAT2):
        nvr = 8 if (q + 1) * 128 <= NPW else (NPW - q * 128 + 15) // 16

        def vreg_body(kk, _, q=q):
            col = kk * 16
            f = q * 128 + col
            i_loc = f + _lane()
            acc = jnp.zeros((16,), jnp.float32)
            for d in range(3):
                pd = plsc.load_gather(pc_v, [i_loc * 3 + d])
                td = plsc.load_gather(tc_v, [i_loc * 3 + d])
                acc = acc + (pd - td) * (pd - td)
            ta = ta_v[pl.ds(f, 16)]
            ca = _ce_flat(pa_v, i_loc, ta, NCLS_A)
            tch = tch_v[pl.ds(f, 16)]
            cc = _ce_flat(pch_v, i_loc, tch, NCLS_C)
            bsum = bp0s_v[pl.ds(f, 16)] + bp1s_v[pl.ds(f, 16)]
            bcnt = bp0c_v[pl.ds(f, 16)] + bp1c_v[pl.ds(f, 16)]
            bv = 0.5 * bsum / jnp.maximum(bcnt, 1.0)
            g = bt_v[pl.ds(f, 16)]
            dest = jnp.where(base + i_loc >= vstart, g, G_JUNK)
            idx_v[q, pl.ds(col, 16)] = dest
            r_v[pl.ds(f, 16)] = acc * (1.0 / 3.0)
            a_v[pl.ds(f, 16)] = ca
            c_v[pl.ds(f, 16)] = cc
            b_v[pl.ds(f, 16)] = bv
            return 0

        lax.fori_loop(0, nvr, vreg_body, 0)

    srcs = [r_v, a_v, c_v, b_v, ones_v]
    copies = []
    for q in range(SCAT2):
        for src, tbl in zip(srcs, gtabs):
            copies.append(pltpu.async_copy(
                src.at[pl.ds(q * 128, 128)],
                tbl.at[idx_v.at[q]], sem, add=True))
    for c in copies:
        c.wait()
    plsc.subcore_barrier()
    for tn, tbl in enumerate(gtabs):
        pltpu.sync_copy(tbl.at[pl.ds(sid * ROWS2_PT, ROWS2_PT)],
                        zb_v.at[pl.ds(0, ROWS2_PT)])
        pltpu.sync_copy(
            zb_v.at[pl.ds(0, ROWS2_PT)],
            out_hbm.at[pl.ds((cid * 5 + tn) * GTAB_ROWS + sid * ROWS2_PT,
                             ROWS2_PT)])


def _k3_body(gp_hbm, w_hbm, out_hbm, t0_v, t1_v, cnt_v, w_v, ob_v, sem):
    cid = lax.axis_index("c")
    sid = lax.axis_index("s")

    @pl.when(jnp.logical_and(cid == 0, sid == 0))
    def _():
        o1 = 5 * GTAB_ROWS
        pltpu.async_copy(gp_hbm.at[pl.ds(4 * GTAB_ROWS, B)], t0_v, sem).wait()
        pltpu.async_copy(gp_hbm.at[pl.ds(o1 + 4 * GTAB_ROWS, B)],
                         t1_v, sem).wait()
        pltpu.async_copy(w_hbm, w_v, sem).wait()

        def cnt_body(k, _):
            f = k * 16
            cnt_v[pl.ds(f, 16)] = jnp.maximum(
                t0_v[pl.ds(f, 16)] + t1_v[pl.ds(f, 16)], 1.0)
            return 0

        lax.fori_loop(0, B // 16, cnt_body, 0)

        lane = _lane()
        out = jnp.zeros((16,), jnp.float32)
        for tbl in range(4):
            pltpu.async_copy(gp_hbm.at[pl.ds(tbl * GTAB_ROWS, B)],
                             t0_v, sem).wait()
            pltpu.async_copy(gp_hbm.at[pl.ds(o1 + tbl * GTAB_ROWS, B)],
                             t1_v, sem).wait()

            def body(k, acc):
                f = k * 16
                tot = t0_v[pl.ds(f, 16)] + t1_v[pl.ds(f, 16)]
                return acc + w_v[pl.ds(f, 16)] * tot / cnt_v[pl.ds(f, 16)]

            acc = lax.fori_loop(0, B // 16, body, jnp.zeros((16,),
                                                            jnp.float32))
            li = jnp.sum(acc, axis=0)
            out = out + jnp.where(lane == tbl, li, 0.0)
        ob_v[...] = out
        pltpu.sync_copy(ob_v, out_hbm)


_mesh = plsc.VectorSubcoreMesh(core_axis_name="c", subcore_axis_name="s")
_cparams = pltpu.CompilerParams(needs_layout_passes=False)

_k1 = functools.partial(
    pl.kernel,
    out_type=jax.ShapeDtypeStruct((NC * 2 * ATAB_ROWS,), jnp.float32),
    mesh=_mesh,
    compiler_params=_cparams,
    scratch_types=[
        pltpu.VMEM((NCLS_B * CHP,), jnp.float32),
        pltpu.VMEM((SCAT1 * 128,), jnp.int32),
        pltpu.VMEM((SCAT1 * 128,), jnp.int32),
        pltpu.VMEM((SCAT1 * 128,), jnp.float32),
        pltpu.VMEM((SCAT1, 128), jnp.int32),
        pltpu.VMEM((ROWS1_PT + 8,), jnp.float32),
        pltpu.VMEM_SHARED((ATAB_ROWS,), jnp.float32),
        pltpu.VMEM_SHARED((ATAB_ROWS,), jnp.float32),
        pltpu.SemaphoreType.DMA,
    ],
)(_k1_body)

_k2 = functools.partial(
    pl.kernel,
    out_type=jax.ShapeDtypeStruct((NC * 5 * GTAB_ROWS,), jnp.float32),
    mesh=_mesh,
    compiler_params=_cparams,
    scratch_types=[
        pltpu.VMEM((NPW * 3,), jnp.float32),
        pltpu.VMEM((NPW * 3,), jnp.float32),
        pltpu.VMEM((NPW * NCLS_A,), jnp.float32),
        pltpu.VMEM((NPW,), jnp.int32),
        pltpu.VMEM((NPW * NCLS_C,), jnp.float32),
        pltpu.VMEM((NPW,), jnp.int32),
        pltpu.VMEM((NPW,), jnp.int32),
        pltpu.VMEM((NPW,), jnp.float32),
        pltpu.VMEM((NPW,), jnp.float32),
        pltpu.VMEM((NPW,), jnp.float32),
        pltpu.VMEM((NPW,), jnp.float32),
        pltpu.VMEM((SCAT2 * 128,), jnp.float32),
        pltpu.VMEM((SCAT2 * 128,), jnp.float32),
        pltpu.VMEM((SCAT2 * 128,), jnp.float32),
        pltpu.VMEM((SCAT2 * 128,), jnp.float32),
        pltpu.VMEM((SCAT2 * 128,), jnp.float32),
        pltpu.VMEM((SCAT2, 128), jnp.int32),
        pltpu.VMEM((80,), jnp.float32),
        [pltpu.VMEM_SHARED((GTAB_ROWS,), jnp.float32) for _ in range(5)],
        pltpu.SemaphoreType.DMA,
    ],
)(_k2_body)

_k3 = functools.partial(
    pl.kernel,
    out_type=jax.ShapeDtypeStruct((16,), jnp.float32),
    mesh=_mesh,
    compiler_params=_cparams,
    scratch_types=[
        pltpu.VMEM((B,), jnp.float32),
        pltpu.VMEM((B,), jnp.float32),
        pltpu.VMEM((B,), jnp.float32),
        pltpu.VMEM((B,), jnp.float32),
        pltpu.VMEM((16,), jnp.float32),
        pltpu.SemaphoreType.DMA,
    ],
)(_k3_body)


def kernel(pred_coords, true_coords, pred_atoms, true_atoms, pred_charges,
           true_charges, pred_bonds, true_bonds, batch,
           bond_aggregation_index, weights):
    pbt = pred_bonds.T.reshape(-1)
    atab = _k1(pbt, true_bonds, bond_aggregation_index)
    gtab = _k2(pred_coords.reshape(-1), true_coords.reshape(-1),
               pred_atoms.reshape(-1), true_atoms,
               pred_charges.reshape(-1), true_charges, batch, atab)
    out = _k3(gtab, weights)
    return out[:4]


# speedup vs baseline: 3.9748x; 1.2334x over previous
import functools

import jax
import jax.numpy as jnp
from jax import lax
from jax.experimental import pallas as pl
from jax.experimental.pallas import tpu as pltpu
from jax.experimental.pallas import tpu_sc as plsc

B = 1024
N = 50000
E = 800000
NCLS_A = 16
NCLS_C = 6
NCLS_B = 5

NC = 2
NS = 16
NW = NC * NS

EPW = E // NW
CH = 1000
NCHUNK = EPW // CH
CHP = CH + 16
SCAT1 = 8
ATAB_ROWS = 50048
A_JUNK = N
ROWS1_PT = ATAB_ROWS // NS

NPW = 1568
SCAT2 = 13
GTAB_ROWS = 1152
G_JUNK = B
ROWS2_PT = GTAB_ROWS // NS

LN2 = 0.6931471805599453


def _ln(s):
    bits = plsc.bitcast(s, jnp.int32)
    e = (bits >> 23) - 127
    m = plsc.bitcast((bits & 0x007FFFFF) | 0x3F800000, jnp.float32)
    z = (m - 1.0) / (m + 1.0)
    z2 = z * z
    p = 1.0 / 7.0 + z2 * (1.0 / 9.0)
    p = 1.0 / 5.0 + z2 * p
    p = 1.0 / 3.0 + z2 * p
    atanh = z * (1.0 + z2 * p)
    return e.astype(jnp.float32) * LN2 + 2.0 * atanh


def _lane():
    return lax.iota(jnp.int32, 16)


def _ce_flat(buf, i_loc, t, ncls, stride=None):
    base = i_loc * (stride or ncls)
    xs = [plsc.load_gather(buf, [base + c]) for c in range(ncls)]
    m = xs[0]
    for x in xs[1:]:
        m = jnp.maximum(m, x)
    s = jnp.exp(xs[0] - m)
    for x in xs[1:]:
        s = s + jnp.exp(x - m)
    xt = plsc.load_gather(buf, [base + t])
    return _ln(s) + m - xt


def _fill(buf, n, value):
    def body(k, _):
        buf[pl.ds(k * 16, 16)] = jnp.full((16,), value, jnp.float32)
        return 0
    lax.fori_loop(0, n // 16, body, 0)


def _k1_body(pb_hbm, tb_hbm, ai_hbm, out_hbm,
             lg_v, tb_v, ai_v, val_v, idx_v, zb_v,
             atab_s, atab_c, sem):
    cid = lax.axis_index("c")
    sid = lax.axis_index("s")
    wid = sid * NC + cid
    ebase = wid * EPW

    _fill(zb_v, ROWS1_PT + 8, 0.0)
    pltpu.sync_copy(zb_v.at[pl.ds(0, ROWS1_PT)],
                    atab_s.at[pl.ds(sid * ROWS1_PT, ROWS1_PT)])
    pltpu.sync_copy(zb_v.at[pl.ds(0, ROWS1_PT)],
                    atab_c.at[pl.ds(sid * ROWS1_PT, ROWS1_PT)])
    _fill(zb_v, SCAT1 * 128, 1.0)
    jnk = jnp.full((16,), A_JUNK, dtype=jnp.int32)
    for col in range((CH % 128 + 15) // 16 * 16, 128, 16):
        idx_v[CH // 128, pl.ds(col, 16)] = jnk
    plsc.subcore_barrier()

    def chunk_body(j, _):
        off = ebase + j * CH
        in_copies = []
        for c in range(NCLS_B):
            in_copies.append(pltpu.async_copy(
                pb_hbm.at[pl.ds(c * E + off, CH)],
                lg_v.at[pl.ds(c * CHP, CH)], sem))
        in_copies.append(pltpu.async_copy(
            tb_hbm.at[pl.ds(off, CH)], tb_v.at[pl.ds(0, CH)], sem))
        in_copies.append(pltpu.async_copy(
            ai_hbm.at[pl.ds(off, CH)], ai_v.at[pl.ds(0, CH)], sem))
        for c in in_copies:
            c.wait()

        for q in range(SCAT1):
            nvr = 8 if (q + 1) * 128 <= CH else (CH - q * 128 + 15) // 16

            def vreg_body(kk, _, q=q):
                col = kk * 16
                f = q * 128 + col
                i_loc = f + _lane()
                valid = i_loc < CH
                t = tb_v[pl.ds(f, 16)]
                xs = [lg_v[pl.ds(c * CHP + f, 16)] for c in range(NCLS_B)]
                m = xs[0]
                for x in xs[1:]:
                    m = jnp.maximum(m, x)
                s = jnp.exp(xs[0] - m)
                for x in xs[1:]:
                    s = s + jnp.exp(x - m)
                xt = xs[0]
                for c in range(1, NCLS_B):
                    xt = jnp.where(t == c, xs[c], xt)
                ce = _ln(s) + m - xt
                g = ai_v[pl.ds(f, 16)]
                dest = jnp.where(valid, g, A_JUNK)
                idx_v[q, pl.ds(col, 16)] = dest
                val_v[pl.ds(f, 16)] = ce
                return 0

            lax.fori_loop(0, nvr, vreg_body, 0)

        copies = []
        for q in range(SCAT1):
            copies.append(pltpu.async_copy(
                val_v.at[pl.ds(q * 128, 128)],
                atab_s.at[idx_v.at[q]], sem, add=True))
            copies.append(pltpu.async_copy(
                zb_v.at[pl.ds(q * 128, 128)],
                atab_c.at[idx_v.at[q]], sem, add=True))
        for c in copies:
            c.wait()
        return 0

    lax.fori_loop(0, NCHUNK, chunk_body, 0)
    plsc.subcore_barrier()
    pltpu.sync_copy(atab_s.at[pl.ds(sid * ROWS1_PT, ROWS1_PT)],
                    zb_v.at[pl.ds(0, ROWS1_PT)])
    pltpu.sync_copy(
        zb_v.at[pl.ds(0, ROWS1_PT)],
        out_hbm.at[pl.ds(cid * 2 * ATAB_ROWS + sid * ROWS1_PT, ROWS1_PT)])
    pltpu.sync_copy(atab_c.at[pl.ds(sid * ROWS1_PT, ROWS1_PT)],
                    zb_v.at[pl.ds(0, ROWS1_PT)])
    pltpu.sync_copy(
        zb_v.at[pl.ds(0, ROWS1_PT)],
        out_hbm.at[pl.ds((cid * 2 + 1) * ATAB_ROWS + sid * ROWS1_PT,
                         ROWS1_PT)])


def _k2_body(pc_hbm, tc_hbm, pa_hbm, ta_hbm, pch_hbm, tch_hbm, bt_hbm,
             bp_hbm, out_hbm,
             pc_v, tc_v, pa_v, ta_v, pch_v, tch_v, bt_v,
             bp0s_v, bp0c_v, bp1s_v, bp1c_v,
             r_v, a_v, c_v, b_v, ones_v, idx_v, zb_v, gtabs, sem):
    cid = lax.axis_index("c")
    sid = lax.axis_index("s")
    wid = sid * NC + cid
    vstart = wid * NPW
    base = jnp.minimum(vstart, N - NPW)

    _fill(zb_v, 80, 0.0)
    for tbl in gtabs:
        pltpu.sync_copy(zb_v.at[pl.ds(0, ROWS2_PT)],
                        tbl.at[pl.ds(sid * ROWS2_PT, ROWS2_PT)])
    _fill(ones_v, SCAT2 * 128, 1.0)
    jnk = jnp.full((16,), G_JUNK, dtype=jnp.int32)
    for col in range(NPW % 128, 128, 16):
        idx_v[NPW // 128, pl.ds(col, 16)] = jnk
    plsc.subcore_barrier()

    copies = [
        pltpu.async_copy(pc_hbm.at[pl.ds(base * 3, NPW * 3)], pc_v, sem),
        pltpu.async_copy(tc_hbm.at[pl.ds(base * 3, NPW * 3)], tc_v, sem),
        pltpu.async_copy(pa_hbm.at[pl.ds(base * NCLS_A, NPW * NCLS_A)],
                         pa_v, sem),
        pltpu.async_copy(ta_hbm.at[pl.ds(base, NPW)], ta_v, sem),
        pltpu.async_copy(pch_hbm.at[pl.ds(base * NCLS_C, NPW * NCLS_C)],
                         pch_v, sem),
        pltpu.async_copy(tch_hbm.at[pl.ds(base, NPW)], tch_v, sem),
        pltpu.async_copy(bt_hbm.at[pl.ds(base, NPW)], bt_v, sem),
        pltpu.async_copy(bp_hbm.at[pl.ds(base, NPW)], bp0s_v, sem),
        pltpu.async_copy(bp_hbm.at[pl.ds(ATAB_ROWS + base, NPW)],
                         bp0c_v, sem),
        pltpu.async_copy(bp_hbm.at[pl.ds(2 * ATAB_ROWS + base, NPW)],
                         bp1s_v, sem),
        pltpu.async_copy(bp_hbm.at[pl.ds(3 * ATAB_ROWS + base, NPW)],
                         bp1c_v, sem),
    ]
    for c in copies:
        c.wait()

    for q in range(SCAT2):
        nvr = 8 if (q + 1) * 128 <= NPW else (NPW - q * 128 + 15) // 16

        def vreg_body(kk, _, q=q):
            col = kk * 16
            f = q * 128 + col
            i_loc = f + _lane()
            acc = jnp.zeros((16,), jnp.float32)
            for d in range(3):
                pd = plsc.load_gather(pc_v, [i_loc * 3 + d])
                td = plsc.load_gather(tc_v, [i_loc * 3 + d])
                acc = acc + (pd - td) * (pd - td)
            ta = ta_v[pl.ds(f, 16)]
            ca = _ce_flat(pa_v, i_loc, ta, NCLS_A)
            tch = tch_v[pl.ds(f, 16)]
            cc = _ce_flat(pch_v, i_loc, tch, NCLS_C)
            bsum = bp0s_v[pl.ds(f, 16)] + bp1s_v[pl.ds(f, 16)]
            bcnt = bp0c_v[pl.ds(f, 16)] + bp1c_v[pl.ds(f, 16)]
            bv = 0.5 * bsum / jnp.maximum(bcnt, 1.0)
            g = bt_v[pl.ds(f, 16)]
            dest = jnp.where(base + i_loc >= vstart, g, G_JUNK)
            idx_v[q, pl.ds(col, 16)] = dest
            r_v[pl.ds(f, 16)] = acc * (1.0 / 3.0)
            a_v[pl.ds(f, 16)] = ca
            c_v[pl.ds(f, 16)] = cc
            b_v[pl.ds(f, 16)] = bv
            return 0

        lax.fori_loop(0, nvr, vreg_body, 0)

    srcs = [r_v, a_v, c_v, b_v, ones_v]
    copies = []
    for q in range(SCAT2):
        for src, tbl in zip(srcs, gtabs):
            copies.append(pltpu.async_copy(
                src.at[pl.ds(q * 128, 128)],
                tbl.at[idx_v.at[q]], sem, add=True))
    for c in copies:
        c.wait()
    plsc.subcore_barrier()
    for tn, tbl in enumerate(gtabs):
        pltpu.sync_copy(tbl.at[pl.ds(sid * ROWS2_PT, ROWS2_PT)],
                        zb_v.at[pl.ds(0, ROWS2_PT)])
        pltpu.sync_copy(
            zb_v.at[pl.ds(0, ROWS2_PT)],
            out_hbm.at[pl.ds((cid * 5 + tn) * GTAB_ROWS + sid * ROWS2_PT,
                             ROWS2_PT)])


def _k3_body(gp_hbm, w_hbm, out_hbm, t0_v, t1_v, cnt_v, w_v, ob_v, sem):
    cid = lax.axis_index("c")
    sid = lax.axis_index("s")

    @pl.when(jnp.logical_and(cid == 0, sid == 0))
    def _():
        o1 = 5 * GTAB_ROWS
        pltpu.async_copy(gp_hbm.at[pl.ds(4 * GTAB_ROWS, B)], t0_v, sem).wait()
        pltpu.async_copy(gp_hbm.at[pl.ds(o1 + 4 * GTAB_ROWS, B)],
                         t1_v, sem).wait()
        pltpu.async_copy(w_hbm, w_v, sem).wait()

        def cnt_body(k, _):
            f = k * 16
            cnt_v[pl.ds(f, 16)] = jnp.maximum(
                t0_v[pl.ds(f, 16)] + t1_v[pl.ds(f, 16)], 1.0)
            return 0

        lax.fori_loop(0, B // 16, cnt_body, 0)

        lane = _lane()
        out = jnp.zeros((16,), jnp.float32)
        for tbl in range(4):
            pltpu.async_copy(gp_hbm.at[pl.ds(tbl * GTAB_ROWS, B)],
                             t0_v, sem).wait()
            pltpu.async_copy(gp_hbm.at[pl.ds(o1 + tbl * GTAB_ROWS, B)],
                             t1_v, sem).wait()

            def body(k, acc):
                f = k * 16
                tot = t0_v[pl.ds(f, 16)] + t1_v[pl.ds(f, 16)]
                return acc + w_v[pl.ds(f, 16)] * tot / cnt_v[pl.ds(f, 16)]

            acc = lax.fori_loop(0, B // 16, body, jnp.zeros((16,),
                                                            jnp.float32))
            li = jnp.sum(acc, axis=0)
            out = out + jnp.where(lane == tbl, li, 0.0)
        ob_v[...] = out
        pltpu.sync_copy(ob_v, out_hbm)


_mesh = plsc.VectorSubcoreMesh(core_axis_name="c", subcore_axis_name="s")
_cparams = pltpu.CompilerParams(needs_layout_passes=False)

_k1 = functools.partial(
    pl.kernel,
    out_type=jax.ShapeDtypeStruct((NC * 2 * ATAB_ROWS,), jnp.float32),
    mesh=_mesh,
    compiler_params=_cparams,
    scratch_types=[
        pltpu.VMEM((NCLS_B * CHP,), jnp.float32),
        pltpu.VMEM((SCAT1 * 128,), jnp.int32),
        pltpu.VMEM((SCAT1 * 128,), jnp.int32),
        pltpu.VMEM((SCAT1 * 128,), jnp.float32),
        pltpu.VMEM((SCAT1, 128), jnp.int32),
        pltpu.VMEM((ROWS1_PT + 8,), jnp.float32),
        pltpu.VMEM_SHARED((ATAB_ROWS,), jnp.float32),
        pltpu.VMEM_SHARED((ATAB_ROWS,), jnp.float32),
        pltpu.SemaphoreType.DMA,
    ],
)(_k1_body)

_k2 = functools.partial(
    pl.kernel,
    out_type=jax.ShapeDtypeStruct((NC * 5 * GTAB_ROWS,), jnp.float32),
    mesh=_mesh,
    compiler_params=_cparams,
    scratch_types=[
        pltpu.VMEM((NPW * 3,), jnp.float32),
        pltpu.VMEM((NPW * 3,), jnp.float32),
        pltpu.VMEM((NPW * NCLS_A,), jnp.float32),
        pltpu.VMEM((NPW,), jnp.int32),
        pltpu.VMEM((NPW * NCLS_C,), jnp.float32),
        pltpu.VMEM((NPW,), jnp.int32),
        pltpu.VMEM((NPW,), jnp.int32),
        pltpu.VMEM((NPW,), jnp.float32),
        pltpu.VMEM((NPW,), jnp.float32),
        pltpu.VMEM((NPW,), jnp.float32),
        pltpu.VMEM((NPW,), jnp.float32),
        pltpu.VMEM((SCAT2 * 128,), jnp.float32),
        pltpu.VMEM((SCAT2 * 128,), jnp.float32),
        pltpu.VMEM((SCAT2 * 128,), jnp.float32),
        pltpu.VMEM((SCAT2 * 128,), jnp.float32),
        pltpu.VMEM((SCAT2 * 128,), jnp.float32),
        pltpu.VMEM((SCAT2, 128), jnp.int32),
        pltpu.VMEM((80,), jnp.float32),
        [pltpu.VMEM_SHARED((GTAB_ROWS,), jnp.float32) for _ in range(5)],
        pltpu.SemaphoreType.DMA,
    ],
)(_k2_body)

_k3 = functools.partial(
    pl.kernel,
    out_type=jax.ShapeDtypeStruct((16,), jnp.float32),
    mesh=_mesh,
    compiler_params=_cparams,
    scratch_types=[
        pltpu.VMEM((B,), jnp.float32),
        pltpu.VMEM((B,), jnp.float32),
        pltpu.VMEM((B,), jnp.float32),
        pltpu.VMEM((B,), jnp.float32),
        pltpu.VMEM((16,), jnp.float32),
        pltpu.SemaphoreType.DMA,
    ],
)(_k3_body)


def kernel(pred_coords, true_coords, pred_atoms, true_atoms, pred_charges,
           true_charges, pred_bonds, true_bonds, batch,
           bond_aggregation_index, weights):
    pbt = pred_bonds.T.reshape(-1)
    atab = _k1(pbt, true_bonds, bond_aggregation_index)
    gtab = _k2(pred_coords.reshape(-1), true_coords.reshape(-1),
               pred_atoms.reshape(-1), true_atoms,
               pred_charges.reshape(-1), true_charges, batch, atab)
    out = _k3(gtab, weights)
    return out[:4]


# speedup vs baseline: 3.9769x; 1.0005x over previous
import functools

import jax
import jax.numpy as jnp
from jax import lax
from jax.experimental import pallas as pl
from jax.experimental.pallas import tpu as pltpu
from jax.experimental.pallas import tpu_sc as plsc

B = 1024
N = 50000
E = 800000
NCLS_A = 16
NCLS_C = 6
NCLS_B = 5

NC = 2
NS = 16
NW = NC * NS

EPW = E // NW
CH = 1000
NCHUNK = EPW // CH
CHP = CH + 16
SCAT1 = 8
ATAB_ROWS = 50048
A_JUNK = N
ROWS1_PT = ATAB_ROWS // NS

NPW = 1568
SCAT2 = 13
GTAB_ROWS = 1152
G_JUNK = B
ROWS2_PT = GTAB_ROWS // NS

LN2 = 0.6931471805599453


def _ln(s):
    bits = plsc.bitcast(s, jnp.int32)
    e = (bits >> 23) - 127
    m = plsc.bitcast((bits & 0x007FFFFF) | 0x3F800000, jnp.float32)
    z = (m - 1.0) / (m + 1.0)
    z2 = z * z
    p = 1.0 / 7.0 + z2 * (1.0 / 9.0)
    p = 1.0 / 5.0 + z2 * p
    p = 1.0 / 3.0 + z2 * p
    atanh = z * (1.0 + z2 * p)
    return e.astype(jnp.float32) * LN2 + 2.0 * atanh


def _lane():
    return lax.iota(jnp.int32, 16)


def _ce_flat(buf, i_loc, t, ncls, stride=None):
    base = i_loc * (stride or ncls)
    xs = [plsc.load_gather(buf, [base + c]) for c in range(ncls)]
    m = xs[0]
    for x in xs[1:]:
        m = jnp.maximum(m, x)
    s = jnp.exp(xs[0] - m)
    for x in xs[1:]:
        s = s + jnp.exp(x - m)
    xt = plsc.load_gather(buf, [base + t])
    return _ln(s) + m - xt


def _fill(buf, n, value):
    def body(k, _):
        buf[pl.ds(k * 16, 16)] = jnp.full((16,), value, jnp.float32)
        return 0
    lax.fori_loop(0, n // 16, body, 0)


def _k1_body(pb_hbm, tb_hbm, ai_hbm, out_hbm,
             lg_v, tb_v, ai_v, val_v, idx_v, zb_v,
             atab_s, atab_c, sem, sem_in):
    cid = lax.axis_index("c")
    sid = lax.axis_index("s")
    wid = sid * NC + cid
    ebase = wid * EPW

    _fill(zb_v, ROWS1_PT + 8, 0.0)
    pltpu.sync_copy(zb_v.at[pl.ds(0, ROWS1_PT)],
                    atab_s.at[pl.ds(sid * ROWS1_PT, ROWS1_PT)])
    pltpu.sync_copy(zb_v.at[pl.ds(0, ROWS1_PT)],
                    atab_c.at[pl.ds(sid * ROWS1_PT, ROWS1_PT)])
    _fill(zb_v, SCAT1 * 128, 1.0)
    jnk = jnp.full((16,), A_JUNK, dtype=jnp.int32)
    for col in range((CH % 128 + 15) // 16 * 16, 128, 16):
        idx_v[CH // 128, pl.ds(col, 16)] = jnk
    plsc.subcore_barrier()

    LGB = NCLS_B * CHP
    TBP = SCAT1 * 128

    def issue(j, b):
        off = ebase + j * CH
        for c in range(NCLS_B):
            pltpu.async_copy(pb_hbm.at[pl.ds(c * E + off, CH)],
                             lg_v.at[pl.ds(b * LGB + c * CHP, CH)], sem_in)
        pltpu.async_copy(tb_hbm.at[pl.ds(off, CH)],
                         tb_v.at[pl.ds(b * TBP, CH)], sem_in)
        pltpu.async_copy(ai_hbm.at[pl.ds(off, CH)],
                         ai_v.at[pl.ds(b * TBP, CH)], sem_in)

    def drain(b):
        for c in range(NCLS_B):
            pltpu.make_async_copy(
                pb_hbm.at[pl.ds(0, CH)],
                lg_v.at[pl.ds(b * LGB + c * CHP, CH)], sem_in).wait()
        pltpu.make_async_copy(tb_hbm.at[pl.ds(0, CH)],
                              tb_v.at[pl.ds(b * TBP, CH)], sem_in).wait()
        pltpu.make_async_copy(ai_hbm.at[pl.ds(0, CH)],
                              ai_v.at[pl.ds(b * TBP, CH)], sem_in).wait()

    def compute(b):
        for q in range(SCAT1):
            nvr = 8 if (q + 1) * 128 <= CH else (CH - q * 128 + 15) // 16

            def vreg_body(kk, _, q=q):
                col = kk * 16
                f = q * 128 + col
                i_loc = f + _lane()
                valid = i_loc < CH
                t = tb_v[pl.ds(b * TBP + f, 16)]
                xs = [lg_v[pl.ds(b * LGB + c * CHP + f, 16)]
                      for c in range(NCLS_B)]
                m = xs[0]
                for x in xs[1:]:
                    m = jnp.maximum(m, x)
                s = jnp.exp(xs[0] - m)
                for x in xs[1:]:
                    s = s + jnp.exp(x - m)
                xt = xs[0]
                for c in range(1, NCLS_B):
                    xt = jnp.where(t == c, xs[c], xt)
                ce = _ln(s) + m - xt
                g = ai_v[pl.ds(b * TBP + f, 16)]
                dest = jnp.where(valid, g, A_JUNK)
                idx_v[q, pl.ds(col, 16)] = dest
                val_v[pl.ds(f, 16)] = ce
                return 0

            lax.fori_loop(0, nvr, vreg_body, 0)

        copies = []
        for q in range(SCAT1):
            copies.append(pltpu.async_copy(
                val_v.at[pl.ds(q * 128, 128)],
                atab_s.at[idx_v.at[q]], sem, add=True))
            copies.append(pltpu.async_copy(
                zb_v.at[pl.ds(q * 128, 128)],
                atab_c.at[idx_v.at[q]], sem, add=True))
        for c in copies:
            c.wait()

    issue(0, 0)

    def pair_body(p, _):
        for b in range(2):
            j = 2 * p + b
            drain(b)
            issue(j + 1, 1 - b)
            compute(b)
        return 0

    lax.fori_loop(0, NCHUNK // 2, pair_body, 0)
    drain(0)
    compute(0)
    plsc.subcore_barrier()
    pltpu.sync_copy(atab_s.at[pl.ds(sid * ROWS1_PT, ROWS1_PT)],
                    zb_v.at[pl.ds(0, ROWS1_PT)])
    pltpu.sync_copy(
        zb_v.at[pl.ds(0, ROWS1_PT)],
        out_hbm.at[pl.ds(cid * 2 * ATAB_ROWS + sid * ROWS1_PT, ROWS1_PT)])
    pltpu.sync_copy(atab_c.at[pl.ds(sid * ROWS1_PT, ROWS1_PT)],
                    zb_v.at[pl.ds(0, ROWS1_PT)])
    pltpu.sync_copy(
        zb_v.at[pl.ds(0, ROWS1_PT)],
        out_hbm.at[pl.ds((cid * 2 + 1) * ATAB_ROWS + sid * ROWS1_PT,
                         ROWS1_PT)])


def _k2_body(pc_hbm, tc_hbm, pa_hbm, ta_hbm, pch_hbm, tch_hbm, bt_hbm,
             bp_hbm, out_hbm,
             pc_v, tc_v, pa_v, ta_v, pch_v, tch_v, bt_v,
             bp0s_v, bp0c_v, bp1s_v, bp1c_v,
             r_v, a_v, c_v, b_v, ones_v, idx_v, zb_v, gtabs, sem):
    cid = lax.axis_index("c")
    sid = lax.axis_index("s")
    wid = sid * NC + cid
    vstart = wid * NPW
    base = jnp.minimum(vstart, N - NPW)

    _fill(zb_v, 80, 0.0)
    for tbl in gtabs:
        pltpu.sync_copy(zb_v.at[pl.ds(0, ROWS2_PT)],
                        tbl.at[pl.ds(sid * ROWS2_PT, ROWS2_PT)])
    _fill(ones_v, SCAT2 * 128, 1.0)
    jnk = jnp.full((16,), G_JUNK, dtype=jnp.int32)
    for col in range(NPW % 128, 128, 16):
        idx_v[NPW // 128, pl.ds(col, 16)] = jnk
    plsc.subcore_barrier()

    copies = [
        pltpu.async_copy(pc_hbm.at[pl.ds(base * 3, NPW * 3)], pc_v, sem),
        pltpu.async_copy(tc_hbm.at[pl.ds(base * 3, NPW * 3)], tc_v, sem),
        pltpu.async_copy(pa_hbm.at[pl.ds(base * NCLS_A, NPW * NCLS_A)],
                         pa_v, sem),
        pltpu.async_copy(ta_hbm.at[pl.ds(base, NPW)], ta_v, sem),
        pltpu.async_copy(pch_hbm.at[pl.ds(base * NCLS_C, NPW * NCLS_C)],
                         pch_v, sem),
        pltpu.async_copy(tch_hbm.at[pl.ds(base, NPW)], tch_v, sem),
        pltpu.async_copy(bt_hbm.at[pl.ds(base, NPW)], bt_v, sem),
        pltpu.async_copy(bp_hbm.at[pl.ds(base, NPW)], bp0s_v, sem),
        pltpu.async_copy(bp_hbm.at[pl.ds(ATAB_ROWS + base, NPW)],
                         bp0c_v, sem),
        pltpu.async_copy(bp_hbm.at[pl.ds(2 * ATAB_ROWS + base, NPW)],
                         bp1s_v, sem),
        pltpu.async_copy(bp_hbm.at[pl.ds(3 * ATAB_ROWS + base, NPW)],
                         bp1c_v, sem),
    ]
    for c in copies:
        c.wait()

    for q in range(SCAT2):
        nvr = 8 if (q + 1) * 128 <= NPW else (NPW - q * 128 + 15) // 16

        def vreg_body(kk, _, q=q):
            col = kk * 16
            f = q * 128 + col
            i_loc = f + _lane()
            acc = jnp.zeros((16,), jnp.float32)
            for d in range(3):
                pd = plsc.load_gather(pc_v, [i_loc * 3 + d])
                td = plsc.load_gather(tc_v, [i_loc * 3 + d])
                acc = acc + (pd - td) * (pd - td)
            ta = ta_v[pl.ds(f, 16)]
            ca = _ce_flat(pa_v, i_loc, ta, NCLS_A)
            tch = tch_v[pl.ds(f, 16)]
            cc = _ce_flat(pch_v, i_loc, tch, NCLS_C)
            bsum = bp0s_v[pl.ds(f, 16)] + bp1s_v[pl.ds(f, 16)]
            bcnt = bp0c_v[pl.ds(f, 16)] + bp1c_v[pl.ds(f, 16)]
            bv = 0.5 * bsum / jnp.maximum(bcnt, 1.0)
            g = bt_v[pl.ds(f, 16)]
            dest = jnp.where(base + i_loc >= vstart, g, G_JUNK)
            idx_v[q, pl.ds(col, 16)] = dest
            r_v[pl.ds(f, 16)] = acc * (1.0 / 3.0)
            a_v[pl.ds(f, 16)] = ca
            c_v[pl.ds(f, 16)] = cc
            b_v[pl.ds(f, 16)] = bv
            return 0

        lax.fori_loop(0, nvr, vreg_body, 0)

    srcs = [r_v, a_v, c_v, b_v, ones_v]
    copies = []
    for q in range(SCAT2):
        for src, tbl in zip(srcs, gtabs):
            copies.append(pltpu.async_copy(
                src.at[pl.ds(q * 128, 128)],
                tbl.at[idx_v.at[q]], sem, add=True))
    for c in copies:
        c.wait()
    plsc.subcore_barrier()
    for tn, tbl in enumerate(gtabs):
        pltpu.sync_copy(tbl.at[pl.ds(sid * ROWS2_PT, ROWS2_PT)],
                        zb_v.at[pl.ds(0, ROWS2_PT)])
        pltpu.sync_copy(
            zb_v.at[pl.ds(0, ROWS2_PT)],
            out_hbm.at[pl.ds((cid * 5 + tn) * GTAB_ROWS + sid * ROWS2_PT,
                             ROWS2_PT)])


def _k3_body(gp_hbm, w_hbm, out_hbm, t0_v, t1_v, cnt_v, w_v, ob_v, sem):
    cid = lax.axis_index("c")
    sid = lax.axis_index("s")

    @pl.when(jnp.logical_and(cid == 0, sid == 0))
    def _():
        o1 = 5 * GTAB_ROWS
        pltpu.async_copy(gp_hbm.at[pl.ds(4 * GTAB_ROWS, B)], t0_v, sem).wait()
        pltpu.async_copy(gp_hbm.at[pl.ds(o1 + 4 * GTAB_ROWS, B)],
                         t1_v, sem).wait()
        pltpu.async_copy(w_hbm, w_v, sem).wait()

        def cnt_body(k, _):
            f = k * 16
            cnt_v[pl.ds(f, 16)] = jnp.maximum(
                t0_v[pl.ds(f, 16)] + t1_v[pl.ds(f, 16)], 1.0)
            return 0

        lax.fori_loop(0, B // 16, cnt_body, 0)

        lane = _lane()
        out = jnp.zeros((16,), jnp.float32)
        for tbl in range(4):
            pltpu.async_copy(gp_hbm.at[pl.ds(tbl * GTAB_ROWS, B)],
                             t0_v, sem).wait()
            pltpu.async_copy(gp_hbm.at[pl.ds(o1 + tbl * GTAB_ROWS, B)],
                             t1_v, sem).wait()

            def body(k, acc):
                f = k * 16
                tot = t0_v[pl.ds(f, 16)] + t1_v[pl.ds(f, 16)]
                return acc + w_v[pl.ds(f, 16)] * tot / cnt_v[pl.ds(f, 16)]

            acc = lax.fori_loop(0, B // 16, body, jnp.zeros((16,),
                                                            jnp.float32))
            li = jnp.sum(acc, axis=0)
            out = out + jnp.where(lane == tbl, li, 0.0)
        ob_v[...] = out
        pltpu.sync_copy(ob_v, out_hbm)


_mesh = plsc.VectorSubcoreMesh(core_axis_name="c", subcore_axis_name="s")
_cparams = pltpu.CompilerParams(needs_layout_passes=False)

_k1 = functools.partial(
    pl.kernel,
    out_type=jax.ShapeDtypeStruct((NC * 2 * ATAB_ROWS,), jnp.float32),
    mesh=_mesh,
    compiler_params=_cparams,
    scratch_types=[
        pltpu.VMEM((2 * NCLS_B * CHP,), jnp.float32),
        pltpu.VMEM((2 * SCAT1 * 128,), jnp.int32),
        pltpu.VMEM((2 * SCAT1 * 128,), jnp.int32),
        pltpu.VMEM((SCAT1 * 128,), jnp.float32),
        pltpu.VMEM((SCAT1, 128), jnp.int32),
        pltpu.VMEM((ROWS1_PT + 8,), jnp.float32),
        pltpu.VMEM_SHARED((ATAB_ROWS,), jnp.float32),
        pltpu.VMEM_SHARED((ATAB_ROWS,), jnp.float32),
        pltpu.SemaphoreType.DMA,
        pltpu.SemaphoreType.DMA,
    ],
)(_k1_body)

_k2 = functools.partial(
    pl.kernel,
    out_type=jax.ShapeDtypeStruct((NC * 5 * GTAB_ROWS,), jnp.float32),
    mesh=_mesh,
    compiler_params=_cparams,
    scratch_types=[
        pltpu.VMEM((NPW * 3,), jnp.float32),
        pltpu.VMEM((NPW * 3,), jnp.float32),
        pltpu.VMEM((NPW * NCLS_A,), jnp.float32),
        pltpu.VMEM((NPW,), jnp.int32),
        pltpu.VMEM((NPW * NCLS_C,), jnp.float32),
        pltpu.VMEM((NPW,), jnp.int32),
        pltpu.VMEM((NPW,), jnp.int32),
        pltpu.VMEM((NPW,), jnp.float32),
        pltpu.VMEM((NPW,), jnp.float32),
        pltpu.VMEM((NPW,), jnp.float32),
        pltpu.VMEM((NPW,), jnp.float32),
        pltpu.VMEM((SCAT2 * 128,), jnp.float32),
        pltpu.VMEM((SCAT2 * 128,), jnp.float32),
        pltpu.VMEM((SCAT2 * 128,), jnp.float32),
        pltpu.VMEM((SCAT2 * 128,), jnp.float32),
        pltpu.VMEM((SCAT2 * 128,), jnp.float32),
        pltpu.VMEM((SCAT2, 128), jnp.int32),
        pltpu.VMEM((80,), jnp.float32),
        [pltpu.VMEM_SHARED((GTAB_ROWS,), jnp.float32) for _ in range(5)],
        pltpu.SemaphoreType.DMA,
    ],
)(_k2_body)

_k3 = functools.partial(
    pl.kernel,
    out_type=jax.ShapeDtypeStruct((16,), jnp.float32),
    mesh=_mesh,
    compiler_params=_cparams,
    scratch_types=[
        pltpu.VMEM((B,), jnp.float32),
        pltpu.VMEM((B,), jnp.float32),
        pltpu.VMEM((B,), jnp.float32),
        pltpu.VMEM((B,), jnp.float32),
        pltpu.VMEM((16,), jnp.float32),
        pltpu.SemaphoreType.DMA,
    ],
)(_k3_body)


def kernel(pred_coords, true_coords, pred_atoms, true_atoms, pred_charges,
           true_charges, pred_bonds, true_bonds, batch,
           bond_aggregation_index, weights):
    pbt = pred_bonds.T.reshape(-1)
    atab = _k1(pbt, true_bonds, bond_aggregation_index)
    gtab = _k2(pred_coords.reshape(-1), true_coords.reshape(-1),
               pred_atoms.reshape(-1), true_atoms,
               pred_charges.reshape(-1), true_charges, batch, atab)
    out = _k3(gtab, weights)
    return out[:4]
